# baseline (speedup 1.0000x reference)
import jax
import jax.numpy as jnp
from jax import lax
from jax.experimental import pallas as pl
from jax.experimental.pallas import tpu as pltpu
from jax.experimental.pallas import tpu_sc as plsc

_BATCH = 16384
_HIST = 50
_D = 64
_B = _BATCH * _HIST
_NC, _NS = 2, 16
_NW = _NC * _NS
_BPW = _B // _NW
_C = 128
_NCHUNK = _BPW // _C
_NBUF = 4
_BB = _BATCH // _C


def _body(idx, table, out, idx_v, rows_v, tt_v, *sems):
    sem_g = sems[:_NBUF]
    sem_s = sems[_NBUF:]
    wid = lax.axis_index("s") * _NC + lax.axis_index("c")
    g0 = wid * _NCHUNK

    pltpu.sync_copy(idx.at[wid], idx_v)

    def start_gather(c, b):
        pltpu.async_copy(table.at[idx_v.at[c]], rows_v.at[b], sem_g[b])

    def wait_gather(c, b):
        pltpu.make_async_copy(table.at[idx_v.at[c]], rows_v.at[b],
                              sem_g[b]).wait()

    def out_slice(c):
        g = g0 + c
        h = g // _BB
        bb = g % _BB
        return out.at[h, :, bb]

    def start_store(c, b):
        pltpu.async_copy(tt_v.at[b], out_slice(c), sem_s[b])

    def wait_store(c, b):
        pltpu.make_async_copy(tt_v.at[b], out_slice(c), sem_s[b]).wait()

    ridx = [lax.iota(jnp.int32, 16) + 16 * lg for lg in range(8)]

    def transpose_chunk(b):
        rows = rows_v.at[b]

        def dtloop(dt, carry):
            base = dt * 8
            for s in range(8):
                col = jnp.full((16,), base + s, jnp.int32)
                for lg in range(8):
                    v = plsc.load_gather(rows, [ridx[lg], col])
                    tt_v[b, dt, s, pl.ds(lg * 16, 16)] = v
            return carry

        lax.fori_loop(0, 8, dtloop, None)

    for b in range(_NBUF):
        start_gather(b, b)

    ngrp = _NCHUNK // _NBUF

    def group(grp, carry):
        for b in range(_NBUF):
            c = grp * _NBUF + b
            wait_gather(c, b)

            @pl.when(grp >= 1)
            def _drain(c=c, b=b):
                wait_store(c - _NBUF, b)

            transpose_chunk(b)
            start_store(c, b)

            @pl.when(grp < ngrp - 1)
            def _next(c=c, b=b):
                start_gather(c + _NBUF, b)

        return carry

    lax.fori_loop(0, ngrp, group, None)

    for b in range(_NBUF):
        wait_store(_NCHUNK - _NBUF + b, b)


def kernel(x, table):
    idx = jnp.swapaxes(x, 0, 1).reshape(_NW, _NCHUNK, _C).astype(jnp.int32)
    mesh = plsc.VectorSubcoreMesh(
        core_axis_name="c", subcore_axis_name="s",
        num_cores=_NC, num_subcores=_NS)
    f = pl.kernel(
        _body,
        out_type=jax.ShapeDtypeStruct((_HIST, _D // 8, _BB, 8, _C),
                                      jnp.float32),
        mesh=mesh,
        scratch_types=[
            pltpu.VMEM((_NCHUNK, _C), jnp.int32),
            pltpu.VMEM((_NBUF, _C, _D), jnp.float32),
            pltpu.VMEM((_NBUF, _D // 8, 8, _C), jnp.float32),
        ] + [pltpu.SemaphoreType.DMA] * (2 * _NBUF),
        compiler_params=pltpu.CompilerParams(use_tc_tiling_on_sc=False,
                                             needs_layout_passes=False),
    )
    out_raw = f(idx, table)
    return out_raw.transpose(2, 4, 0, 1, 3).reshape(_BATCH, _HIST, _D)


# speedup vs baseline: 1.7338x; 1.7338x over previous
import jax
import jax.numpy as jnp
from jax import lax
from jax.experimental import pallas as pl
from jax.experimental.pallas import tpu as pltpu
from jax.experimental.pallas import tpu_sc as plsc

_BATCH = 16384
_HIST = 50
_D = 64
_B = _BATCH * _HIST
_NC, _NS = 2, 16
_NW = _NC * _NS
_BPW = _B // _NW
_C = 128
_NCHUNK = _BPW // _C
_NBUF = 4
_BB = _BATCH // _C


def _body(idx, table, out, idx_v, rows_v, tt_v, *sems):
    sem_g = sems[:_NBUF]
    sem_s = sems[_NBUF:]
    wid = lax.axis_index("s") * _NC + lax.axis_index("c")
    g0 = wid * _NCHUNK

    pltpu.sync_copy(idx.at[wid], idx_v)

    def start_gather(c, b):
        pltpu.async_copy(table.at[idx_v.at[c]], rows_v.at[b], sem_g[b])

    def wait_gather(c, b):
        pltpu.make_async_copy(table.at[idx_v.at[c]], rows_v.at[b],
                              sem_g[b]).wait()

    def out_slice(c):
        g = g0 + c
        h = g // _BB
        bb = g % _BB
        return out.at[h, :, bb]

    def start_store(c, b):
        pltpu.async_copy(tt_v.at[b], out_slice(c), sem_s[b])

    def wait_store(c, b):
        pltpu.make_async_copy(tt_v.at[b], out_slice(c), sem_s[b]).wait()

    ridx = [lax.iota(jnp.int32, 16) + 16 * lg for lg in range(8)]

    iota = lax.iota(jnp.int32, 16)

    def transpose_chunk(b):
        rows = rows_v.at[b]
        tt = tt_v.at[b]

        def dtloop(dt2, carry):
            d0 = dt2 * 16
            for k in range(16):
                cidx = ((iota + k) & 15) + d0
                dt_i = cidx >> 3
                wbase = ((cidx & 7) << 7) + iota
                for lg in range(8):
                    v = plsc.load_gather(rows, [ridx[lg], cidx])
                    plsc.store_scatter(tt, [dt_i, wbase + lg * 16], v)
            return carry

        lax.fori_loop(0, _D // 16, dtloop, None)

    for b in range(_NBUF):
        start_gather(b, b)

    ngrp = _NCHUNK // _NBUF

    def group(grp, carry):
        for b in range(_NBUF):
            c = grp * _NBUF + b
            wait_gather(c, b)

            @pl.when(grp >= 1)
            def _drain(c=c, b=b):
                wait_store(c - _NBUF, b)

            transpose_chunk(b)
            start_store(c, b)

            @pl.when(grp < ngrp - 1)
            def _next(c=c, b=b):
                start_gather(c + _NBUF, b)

        return carry

    lax.fori_loop(0, ngrp, group, None)

    for b in range(_NBUF):
        wait_store(_NCHUNK - _NBUF + b, b)


def kernel(x, table):
    idx = jnp.swapaxes(x, 0, 1).reshape(_NW, _NCHUNK, _C).astype(jnp.int32)
    mesh = plsc.VectorSubcoreMesh(
        core_axis_name="c", subcore_axis_name="s",
        num_cores=_NC, num_subcores=_NS)
    f = pl.kernel(
        _body,
        out_type=jax.ShapeDtypeStruct((_HIST, _D // 8, _BB, 8 * _C),
                                      jnp.float32),
        mesh=mesh,
        scratch_types=[
            pltpu.VMEM((_NCHUNK, _C), jnp.int32),
            pltpu.VMEM((_NBUF, _C, _D), jnp.float32),
            pltpu.VMEM((_NBUF, _D // 8, 8 * _C), jnp.float32),
        ] + [pltpu.SemaphoreType.DMA] * (2 * _NBUF),
        compiler_params=pltpu.CompilerParams(use_tc_tiling_on_sc=False,
                                             needs_layout_passes=False),
    )
    out_raw = f(idx, table).reshape(_HIST, _D // 8, _BB, 8, _C)
    return out_raw.transpose(2, 4, 0, 1, 3).reshape(_BATCH, _HIST, _D)


# speedup vs baseline: 1.9053x; 1.0989x over previous
import jax
import jax.numpy as jnp
from jax import lax
from jax.experimental import pallas as pl
from jax.experimental.pallas import tpu as pltpu
from jax.experimental.pallas import tpu_sc as plsc

_BATCH = 16384
_HIST = 50
_D = 64
_B = _BATCH * _HIST
_NC, _NS = 2, 16
_NW = _NC * _NS
_BPW = _B // _NW
_C = 128
_NCHUNK = _BPW // _C
_NBUF = 2
_BB = _BATCH // _C


def _body(idx, table, out, idx_v, rows_v, tt_v, *sems):
    sem_g = sems[:_NBUF]
    sem_s = sems[_NBUF:]
    wid = lax.axis_index("s") * _NC + lax.axis_index("c")
    g0 = wid * _NCHUNK

    pltpu.sync_copy(idx.at[wid], idx_v)

    def start_gather(c, b):
        pltpu.async_copy(table.at[idx_v.at[c]], rows_v.at[b], sem_g[b])

    def wait_gather(c, b):
        pltpu.make_async_copy(table.at[idx_v.at[c]], rows_v.at[b],
                              sem_g[b]).wait()

    def out_slice(c):
        g = g0 + c
        h = g // _BB
        bb = g % _BB
        return out.at[h, :, bb]

    def start_store(c, b):
        pltpu.async_copy(tt_v.at[b], out_slice(c), sem_s[b])

    def wait_store(c, b):
        pltpu.make_async_copy(tt_v.at[b], out_slice(c), sem_s[b]).wait()

    ridx = [lax.iota(jnp.int32, 16) + 16 * lg for lg in range(8)]

    iota = lax.iota(jnp.int32, 16)

    def transpose_chunk(b):
        rows = rows_v.at[b]
        tt = tt_v.at[b]

        def dtloop(dt2, carry):
            d0 = dt2 * 16
            for k in range(16):
                cidx = ((iota + k) & 15) + d0
                dt_i = cidx >> 3
                wbase = ((cidx & 7) << 7) + iota
                for lg in range(8):
                    v = plsc.load_gather(rows, [ridx[lg], cidx])
                    plsc.store_scatter(tt, [dt_i, wbase + lg * 16], v)
            return carry

        lax.fori_loop(0, _D // 16, dtloop, None)

    for b in range(_NBUF):
        start_gather(b, b)

    ngrp = _NCHUNK // _NBUF

    def group(grp, carry):
        for b in range(_NBUF):
            c = grp * _NBUF + b
            wait_gather(c, b)

            @pl.when(grp >= 1)
            def _drain(c=c, b=b):
                wait_store(c - _NBUF, b)

            transpose_chunk(b)
            start_store(c, b)

            @pl.when(grp < ngrp - 1)
            def _next(c=c, b=b):
                start_gather(c + _NBUF, b)

        return carry

    lax.fori_loop(0, ngrp, group, None)

    for b in range(_NBUF):
        wait_store(_NCHUNK - _NBUF + b, b)


def kernel(x, table):
    idx = jnp.swapaxes(x, 0, 1).reshape(_NW, _NCHUNK, _C).astype(jnp.int32)
    table_p = jnp.pad(table, ((0, 0), (0, _D)))
    mesh = plsc.VectorSubcoreMesh(
        core_axis_name="c", subcore_axis_name="s",
        num_cores=_NC, num_subcores=_NS)
    f = pl.kernel(
        _body,
        out_type=jax.ShapeDtypeStruct((_HIST, _D // 8, _BB, 8 * _C),
                                      jnp.float32),
        mesh=mesh,
        scratch_types=[
            pltpu.VMEM((_NCHUNK, _C), jnp.int32),
            pltpu.VMEM((_NBUF, _C, 2 * _D), jnp.float32),
            pltpu.VMEM((_NBUF, _D // 8, 8 * _C), jnp.float32),
        ] + [pltpu.SemaphoreType.DMA] * (2 * _NBUF),
        compiler_params=pltpu.CompilerParams(use_tc_tiling_on_sc=False,
                                             needs_layout_passes=False),
    )
    out_raw = f(idx, table_p).reshape(_HIST, _D // 8, _BB, 8, _C)
    return out_raw.transpose(2, 4, 0, 1, 3).reshape(_BATCH, _HIST, _D)


# speedup vs baseline: 2.0437x; 1.0727x over previous
import jax
import jax.numpy as jnp
from jax import lax
from jax.experimental import pallas as pl
from jax.experimental.pallas import tpu as pltpu
from jax.experimental.pallas import tpu_sc as plsc

_BATCH = 16384
_HIST = 50
_D = 64
_B = _BATCH * _HIST
_NC, _NS = 2, 16
_NW = _NC * _NS
_BPW = _B // _NW
_C = 128
_NCHUNK = _BPW // _C
_NBUF = 2
_BB = _BATCH // _C


def _body(idx, table, out, idx_v, rows_v, tt_v, *sems):
    sem_g = sems[:_NBUF]
    sem_s = sems[_NBUF:]
    wid = lax.axis_index("s") * _NC + lax.axis_index("c")
    g0 = wid * _NCHUNK

    pltpu.sync_copy(idx.at[wid], idx_v)

    def start_gather(c, b):
        pltpu.async_copy(table.at[idx_v.at[c]], rows_v.at[b], sem_g[b])

    def wait_gather(c, b):
        pltpu.make_async_copy(table.at[idx_v.at[c]], rows_v.at[b],
                              sem_g[b]).wait()

    def out_slice(c):
        g = g0 + c
        h = g // _BB
        bb = g % _BB
        return out.at[h, :, bb]

    def start_store(c, b):
        pltpu.async_copy(tt_v.at[b], out_slice(c), sem_s[b])

    def wait_store(c, b):
        pltpu.make_async_copy(tt_v.at[b], out_slice(c), sem_s[b]).wait()

    ridx = [lax.iota(jnp.int32, 16) + 16 * lg for lg in range(8)]

    iota = lax.iota(jnp.int32, 16)

    def transpose_chunk(b):
        rows = rows_v.at[b]
        tt = tt_v.at[b]

        @plsc.parallel_loop(0, _D // 16, unroll=2)
        def dtloop(dt2):
            d0 = dt2 * 16
            for k in range(16):
                cidx = ((iota + k) & 15) + d0
                dt_i = cidx >> 3
                wbase = ((cidx & 7) << 7) + iota
                for lg in range(8):
                    v = plsc.load_gather(rows, [ridx[lg], cidx])
                    plsc.store_scatter(tt, [dt_i, wbase + lg * 16], v)

    for b in range(_NBUF):
        start_gather(b, b)

    ngrp = _NCHUNK // _NBUF

    def group(grp, carry):
        for b in range(_NBUF):
            c = grp * _NBUF + b
            wait_gather(c, b)

            @pl.when(grp >= 1)
            def _drain(c=c, b=b):
                wait_store(c - _NBUF, b)

            transpose_chunk(b)
            start_store(c, b)

            @pl.when(grp < ngrp - 1)
            def _next(c=c, b=b):
                start_gather(c + _NBUF, b)

        return carry

    lax.fori_loop(0, ngrp, group, None)

    for b in range(_NBUF):
        wait_store(_NCHUNK - _NBUF + b, b)


def kernel(x, table):
    idx = jnp.swapaxes(x, 0, 1).reshape(_NW, _NCHUNK, _C).astype(jnp.int32)
    table_p = jnp.pad(table, ((0, 0), (0, _D)))
    mesh = plsc.VectorSubcoreMesh(
        core_axis_name="c", subcore_axis_name="s",
        num_cores=_NC, num_subcores=_NS)
    f = pl.kernel(
        _body,
        out_type=jax.ShapeDtypeStruct((_HIST, _D // 8, _BB, 8 * _C),
                                      jnp.float32),
        mesh=mesh,
        scratch_types=[
            pltpu.VMEM((_NCHUNK, _C), jnp.int32),
            pltpu.VMEM((_NBUF, _C, 2 * _D), jnp.float32),
            pltpu.VMEM((_NBUF, _D // 8, 8 * _C), jnp.float32),
        ] + [pltpu.SemaphoreType.DMA] * (2 * _NBUF),
        compiler_params=pltpu.CompilerParams(use_tc_tiling_on_sc=False,
                                             needs_layout_passes=False),
    )
    out_raw = f(idx, table_p).reshape(_HIST, _D // 8, _BB, 8, _C)
    return out_raw.transpose(2, 4, 0, 1, 3).reshape(_BATCH, _HIST, _D)
